# baseline (speedup 1.0000x reference)
import functools

import jax
import jax.numpy as jnp
from jax import lax
from jax.experimental import pallas as pl
from jax.experimental.pallas import tpu as pltpu
from jax.experimental.pallas import tpu_sc as plsc

NRAYS = 16384
NSAMP = 524288
HID = 64

IROWS = 128
NWORK = 32
PER_W = NSAMP // NWORK
NCH = PER_W // IROWS

CB = 4096
GRID_B = NSAMP // CB


def _gather_body(table_ref, idx2_ref, out_ref, idx_v, rows_v, sem):
    c = lax.axis_index("c")
    s = lax.axis_index("s")
    wid = c * 16 + s
    pltpu.sync_copy(idx2_ref.at[pl.ds(wid * NCH, NCH)], idx_v)

    def step(j, carry):
        pltpu.async_copy(table_ref.at[idx_v.at[j]], rows_v, sem).wait()
        pltpu.sync_copy(rows_v, out_ref.at[pl.ds(wid * PER_W + j * IROWS, IROWS)])
        return carry

    lax.fori_loop(0, NCH, step, 0)


@functools.cache
def _make_gather():
    return pl.kernel(
        _gather_body,
        out_type=jax.ShapeDtypeStruct((NSAMP, 8), jnp.float32),
        mesh=plsc.VectorSubcoreMesh(core_axis_name="c", subcore_axis_name="s"),
        scratch_types=[
            pltpu.VMEM((NCH, IROWS), jnp.int32),
            pltpu.VMEM((IROWS, 8), jnp.float32),
            pltpu.SemaphoreType.DMA,
        ],
        compiler_params=pltpu.CompilerParams(use_tc_tiling_on_sc=False),
    )


def _scatter_body(vals_ref, idx2_ref, zeros_ref, out_ref, idx_v, vals_v, accum):
    c = lax.axis_index("c")
    s = lax.axis_index("s")
    wid = c * 16 + s

    @pl.when(s == 0)
    def _():
        pltpu.sync_copy(zeros_ref, accum)

    plsc.subcore_barrier()
    pltpu.sync_copy(idx2_ref.at[pl.ds(wid * NCH, NCH)], idx_v)

    def step(j, carry):
        pltpu.sync_copy(vals_ref.at[pl.ds(wid * PER_W + j * IROWS, IROWS)], vals_v)
        pltpu.sync_copy(vals_v, accum.at[idx_v.at[j]], add=True)
        return carry

    lax.fori_loop(0, NCH, step, 0)
    plsc.subcore_barrier()
    rpt = NRAYS // 16
    pltpu.sync_copy(accum.at[pl.ds(s * rpt, rpt)], out_ref.at[c, pl.ds(s * rpt, rpt)])


@functools.cache
def _make_scatter():
    return pl.kernel(
        _scatter_body,
        out_type=jax.ShapeDtypeStruct((2, NRAYS, 8), jnp.float32),
        mesh=plsc.VectorSubcoreMesh(core_axis_name="c", subcore_axis_name="s"),
        scratch_types=[
            pltpu.VMEM((NCH, IROWS), jnp.int32),
            pltpu.VMEM((IROWS, 8), jnp.float32),
            pltpu.VMEM_SHARED((NRAYS, 8), jnp.float32),
        ],
        compiler_params=pltpu.CompilerParams(use_tc_tiling_on_sc=False),
    )


def _main_body(samT, dirs, depT, delT, ridxT, w1t, b1c, w2t, b2c,
               out_ref, ce_ref, cm_ref, cr_ref):
    pid = pl.program_id(0)

    @pl.when(pid == 0)
    def _():
        ce_ref[0, 0] = 0.0
        cm_ref[0, 0] = 0.0
        cr_ref[0, 0] = -1

    carry_e = ce_ref[0, 0]
    carry_m = cm_ref[0, 0]
    carry_r = cr_ref[0, 0]

    feat = jnp.concatenate([samT[...], dirs[...][:3]], axis=0)
    h = lax.dot_general(w1t[...], feat, (((1,), (0,)), ((), ())),
                        preferred_element_type=jnp.float32) + b1c[...]
    h = jnp.maximum(h, 0.0)
    o = lax.dot_general(w2t[...], h, (((1,), (0,)), ((), ())),
                        preferred_element_type=jnp.float32) + b2c[...]
    color = 1.0 / (1.0 + jnp.exp(-o[:3]))
    x = o[3:4]
    sp = jnp.maximum(x, 0.0) + jnp.log(1.0 + jnp.exp(-jnp.abs(x)))
    tau = sp * delT[...]

    lanes = lax.broadcasted_iota(jnp.int32, (1, CB), 1)

    inc = tau
    d = 1
    while d < CB:
        sh = pltpu.roll(inc, d, 1)
        inc = inc + jnp.where(lanes >= d, sh, 0.0)
        d *= 2

    e_excl = (carry_e + inc) - tau

    rr = ridxT[...]
    rprev = pltpu.roll(rr, 1, 1)
    rprev = jnp.where(lanes == 0, carry_r, rprev)
    bnd = rr != rprev
    be = jnp.where(bnd, e_excl, 0.0)
    m = be
    d = 1
    while d < CB:
        sh = pltpu.roll(m, d, 1)
        m = jnp.maximum(m, jnp.where(lanes >= d, sh, 0.0))
        d *= 2
    m = jnp.maximum(m, carry_m)

    excl = e_excl - m
    w = jnp.exp(-excl) - jnp.exp(-(excl + tau))

    out_ref[...] = jnp.concatenate(
        [color * w, depT[...] * w, w], axis=0)

    lastm = lanes == (CB - 1)
    ce_ref[0, 0] = carry_e + jnp.sum(jnp.where(lastm, inc, 0.0))
    cm_ref[0, 0] = jnp.sum(jnp.where(lastm, m, 0.0))
    cr_ref[0, 0] = jnp.sum(jnp.where(lastm, rr, 0))


_main = pl.pallas_call(
    _main_body,
    grid=(GRID_B,),
    in_specs=[
        pl.BlockSpec((3, CB), lambda i: (0, i)),
        pl.BlockSpec((8, CB), lambda i: (0, i)),
        pl.BlockSpec((1, CB), lambda i: (0, i)),
        pl.BlockSpec((1, CB), lambda i: (0, i)),
        pl.BlockSpec((1, CB), lambda i: (0, i)),
        pl.BlockSpec((HID, 6), lambda i: (0, 0)),
        pl.BlockSpec((HID, 1), lambda i: (0, 0)),
        pl.BlockSpec((4, HID), lambda i: (0, 0)),
        pl.BlockSpec((4, 1), lambda i: (0, 0)),
    ],
    out_specs=pl.BlockSpec((5, CB), lambda i: (0, i)),
    out_shape=jax.ShapeDtypeStruct((5, NSAMP), jnp.float32),
    scratch_shapes=[
        pltpu.SMEM((1, 1), jnp.float32),
        pltpu.SMEM((1, 1), jnp.float32),
        pltpu.SMEM((1, 1), jnp.int32),
    ],
)


def _compose_body(p0, p1, rgb_ref, dep_ref, alp_ref, hit_ref):
    S = p0[...] + p1[...]
    alpha = S[:, 4:5]
    rgb_ref[...] = (1.0 - alpha) + alpha * S[:, 0:3]
    dep_ref[...] = S[:, 3:4]
    alp_ref[...] = alpha
    hit_ref[...] = (alpha > 0.0).astype(jnp.int32)


RB = 2048
_compose = pl.pallas_call(
    _compose_body,
    grid=(NRAYS // RB,),
    in_specs=[
        pl.BlockSpec((RB, 8), lambda i: (i, 0)),
        pl.BlockSpec((RB, 8), lambda i: (i, 0)),
    ],
    out_specs=(
        pl.BlockSpec((RB, 3), lambda i: (i, 0)),
        pl.BlockSpec((RB, 1), lambda i: (i, 0)),
        pl.BlockSpec((RB, 1), lambda i: (i, 0)),
        pl.BlockSpec((RB, 1), lambda i: (i, 0)),
    ),
    out_shape=(
        jax.ShapeDtypeStruct((NRAYS, 3), jnp.float32),
        jax.ShapeDtypeStruct((NRAYS, 1), jnp.float32),
        jax.ShapeDtypeStruct((NRAYS, 1), jnp.float32),
        jax.ShapeDtypeStruct((NRAYS, 1), jnp.int32),
    ),
)


def kernel(rays_origins, rays_dirs, ridx, samples, depths, deltas, W1, b1, W2, b2):
    del rays_origins
    ridx32 = ridx.astype(jnp.int32)
    ridx2 = ridx32.reshape(NSAMP // IROWS, IROWS)
    table = jnp.concatenate(
        [rays_dirs, jnp.zeros((NRAYS, 5), jnp.float32)], axis=1)
    dirs8 = _make_gather()(table, ridx2)

    vals_cm = _main(samples.T, dirs8.T, depths.T, deltas.T,
                    ridx32.reshape(1, NSAMP), W1.T,
                    b1.reshape(HID, 1), W2.T, b2.reshape(4, 1))
    vals = jnp.concatenate(
        [vals_cm.T, jnp.zeros((NSAMP, 3), jnp.float32)], axis=1)

    zer = jnp.zeros((NRAYS, 8), jnp.float32)
    parts = _make_scatter()(vals, ridx2, zer)

    rgb, dep, alp, hit32 = _compose(parts[0], parts[1])
    return (rgb, dep, alp, hit32[:, 0] != 0)


# speedup vs baseline: 1.2167x; 1.2167x over previous
import functools

import jax
import jax.numpy as jnp
from jax import lax
from jax.experimental import pallas as pl
from jax.experimental.pallas import tpu as pltpu
from jax.experimental.pallas import tpu_sc as plsc

NRAYS = 16384
NSAMP = 524288
HID = 64

IROWS = 128
NWORK = 32
PER_W = NSAMP // NWORK
NCH = PER_W // IROWS

CB = 4096
GRID_B = NSAMP // CB


def _gather_body(table_ref, idx2_ref, out_ref, idx_v, rows_v, sem):
    c = lax.axis_index("c")
    s = lax.axis_index("s")
    wid = c * 16 + s
    pltpu.sync_copy(idx2_ref.at[pl.ds(wid * NCH, NCH)], idx_v)

    def step(j, carry):
        pltpu.async_copy(table_ref.at[idx_v.at[j]], rows_v, sem).wait()
        pltpu.sync_copy(rows_v, out_ref.at[pl.ds(wid * PER_W + j * IROWS, IROWS)])
        return carry

    lax.fori_loop(0, NCH, step, 0)


@functools.cache
def _make_gather():
    return pl.kernel(
        _gather_body,
        out_type=jax.ShapeDtypeStruct((NSAMP, 8), jnp.float32),
        mesh=plsc.VectorSubcoreMesh(core_axis_name="c", subcore_axis_name="s"),
        scratch_types=[
            pltpu.VMEM((NCH, IROWS), jnp.int32),
            pltpu.VMEM((IROWS, 8), jnp.float32),
            pltpu.SemaphoreType.DMA,
        ],
        compiler_params=pltpu.CompilerParams(use_tc_tiling_on_sc=False),
    )


def _scatter_body(vals_ref, idx2_ref, zeros_ref, out_ref, idx_v, vals_v, accum):
    c = lax.axis_index("c")
    s = lax.axis_index("s")
    wid = c * 16 + s

    @pl.when(s == 0)
    def _():
        pltpu.sync_copy(zeros_ref, accum)

    plsc.subcore_barrier()
    pltpu.sync_copy(idx2_ref.at[pl.ds(wid * NCH, NCH)], idx_v)

    def step(j, carry):
        pltpu.sync_copy(vals_ref.at[pl.ds(wid * PER_W + j * IROWS, IROWS)], vals_v)
        pltpu.sync_copy(vals_v, accum.at[idx_v.at[j]], add=True)
        return carry

    lax.fori_loop(0, NCH, step, 0)
    plsc.subcore_barrier()
    rpt = NRAYS // 16
    pltpu.sync_copy(accum.at[pl.ds(s * rpt, rpt)], out_ref.at[c, pl.ds(s * rpt, rpt)])


@functools.cache
def _make_scatter():
    return pl.kernel(
        _scatter_body,
        out_type=jax.ShapeDtypeStruct((2, NRAYS, 8), jnp.float32),
        mesh=plsc.VectorSubcoreMesh(core_axis_name="c", subcore_axis_name="s"),
        scratch_types=[
            pltpu.VMEM((NCH, IROWS), jnp.int32),
            pltpu.VMEM((IROWS, 8), jnp.float32),
            pltpu.VMEM_SHARED((NRAYS, 8), jnp.float32),
        ],
        compiler_params=pltpu.CompilerParams(use_tc_tiling_on_sc=False),
    )


def _main_body(samT, dirs, depT, delT, ridxT, w1t, b1c, w2t, b2c,
               out_ref, ce_ref, cm_ref, cr_ref):
    pid = pl.program_id(0)

    @pl.when(pid == 0)
    def _():
        ce_ref[0, 0] = 0.0
        cm_ref[0, 0] = 0.0
        cr_ref[0, 0] = -1

    carry_e = ce_ref[0, 0]
    carry_m = cm_ref[0, 0]
    carry_r = cr_ref[0, 0]

    dT = dirs[...].T
    feat = jnp.concatenate([samT[...], dT[:3]], axis=0)
    h = lax.dot_general(w1t[...], feat, (((1,), (0,)), ((), ())),
                        preferred_element_type=jnp.float32) + b1c[...]
    h = jnp.maximum(h, 0.0)
    o = lax.dot_general(w2t[...], h, (((1,), (0,)), ((), ())),
                        preferred_element_type=jnp.float32) + b2c[...]
    color = 1.0 / (1.0 + jnp.exp(-o[:3]))
    x = o[3:4]
    sp = jnp.maximum(x, 0.0) + jnp.log(1.0 + jnp.exp(-jnp.abs(x)))
    tau = sp * delT[...]

    lanes = lax.broadcasted_iota(jnp.int32, (1, CB), 1)

    inc = tau
    d = 1
    while d < CB:
        sh = pltpu.roll(inc, d, 1)
        inc = inc + jnp.where(lanes >= d, sh, 0.0)
        d *= 2

    e_excl = (carry_e + inc) - tau

    rr = ridxT[...]
    rprev = pltpu.roll(rr, 1, 1)
    rprev = jnp.where(lanes == 0, carry_r, rprev)
    bnd = rr != rprev
    be = jnp.where(bnd, e_excl, 0.0)
    m = be
    d = 1
    while d < CB:
        sh = pltpu.roll(m, d, 1)
        m = jnp.maximum(m, jnp.where(lanes >= d, sh, 0.0))
        d *= 2
    m = jnp.maximum(m, carry_m)

    excl = e_excl - m
    w = jnp.exp(-excl) - jnp.exp(-(excl + tau))

    valsT = jnp.concatenate(
        [color * w, depT[...] * w, w, jnp.zeros((3, CB), jnp.float32)], axis=0)
    out_ref[...] = valsT.T

    lastm = lanes == (CB - 1)
    ce_ref[0, 0] = carry_e + jnp.sum(jnp.where(lastm, inc, 0.0))
    cm_ref[0, 0] = jnp.sum(jnp.where(lastm, m, 0.0))
    cr_ref[0, 0] = jnp.sum(jnp.where(lastm, rr, 0))


_main = pl.pallas_call(
    _main_body,
    grid=(GRID_B,),
    in_specs=[
        pl.BlockSpec((3, CB), lambda i: (0, i)),
        pl.BlockSpec((CB, 8), lambda i: (i, 0)),
        pl.BlockSpec((1, CB), lambda i: (0, i)),
        pl.BlockSpec((1, CB), lambda i: (0, i)),
        pl.BlockSpec((1, CB), lambda i: (0, i)),
        pl.BlockSpec((HID, 6), lambda i: (0, 0)),
        pl.BlockSpec((HID, 1), lambda i: (0, 0)),
        pl.BlockSpec((4, HID), lambda i: (0, 0)),
        pl.BlockSpec((4, 1), lambda i: (0, 0)),
    ],
    out_specs=pl.BlockSpec((CB, 8), lambda i: (i, 0)),
    out_shape=jax.ShapeDtypeStruct((NSAMP, 8), jnp.float32),
    scratch_shapes=[
        pltpu.SMEM((1, 1), jnp.float32),
        pltpu.SMEM((1, 1), jnp.float32),
        pltpu.SMEM((1, 1), jnp.int32),
    ],
)


def _compose_body(p0, p1, rgb_ref, dep_ref, alp_ref, hit_ref):
    S = p0[...] + p1[...]
    alpha = S[:, 4:5]
    rgb_ref[...] = (1.0 - alpha) + alpha * S[:, 0:3]
    dep_ref[...] = S[:, 3:4]
    alp_ref[...] = alpha
    hit_ref[...] = (alpha > 0.0).astype(jnp.int32)


RB = 2048
_compose = pl.pallas_call(
    _compose_body,
    grid=(NRAYS // RB,),
    in_specs=[
        pl.BlockSpec((RB, 8), lambda i: (i, 0)),
        pl.BlockSpec((RB, 8), lambda i: (i, 0)),
    ],
    out_specs=(
        pl.BlockSpec((RB, 3), lambda i: (i, 0)),
        pl.BlockSpec((RB, 1), lambda i: (i, 0)),
        pl.BlockSpec((RB, 1), lambda i: (i, 0)),
        pl.BlockSpec((RB, 1), lambda i: (i, 0)),
    ),
    out_shape=(
        jax.ShapeDtypeStruct((NRAYS, 3), jnp.float32),
        jax.ShapeDtypeStruct((NRAYS, 1), jnp.float32),
        jax.ShapeDtypeStruct((NRAYS, 1), jnp.float32),
        jax.ShapeDtypeStruct((NRAYS, 1), jnp.int32),
    ),
)


def kernel(rays_origins, rays_dirs, ridx, samples, depths, deltas, W1, b1, W2, b2):
    del rays_origins
    ridx32 = ridx.astype(jnp.int32)
    ridx2 = ridx32.reshape(NSAMP // IROWS, IROWS)
    table = jnp.concatenate(
        [rays_dirs, jnp.zeros((NRAYS, 5), jnp.float32)], axis=1)
    dirs8 = _make_gather()(table, ridx2)

    vals = _main(samples.T, dirs8, depths.T, deltas.T,
                 ridx32.reshape(1, NSAMP), W1.T,
                 b1.reshape(HID, 1), W2.T, b2.reshape(4, 1))

    zer = jnp.zeros((NRAYS, 8), jnp.float32)
    parts = _make_scatter()(vals, ridx2, zer)

    rgb, dep, alp, hit32 = _compose(parts[0], parts[1])
    return (rgb, dep, alp, hit32[:, 0] != 0)


# speedup vs baseline: 1.4326x; 1.1775x over previous
import functools

import jax
import jax.numpy as jnp
from jax import lax
from jax.experimental import pallas as pl
from jax.experimental.pallas import tpu as pltpu
from jax.experimental.pallas import tpu_sc as plsc

NRAYS = 16384
NSAMP = 524288
HID = 64

NWORK = 32
PER_W = NSAMP // NWORK
SCH = 2048
NSTR = PER_W // SCH

CB = 4096
GRID_B = NSAMP // CB


def _gather_body(table_ref, idx_ref, out_ref, idx_v, rows_v, sems):
    c = lax.axis_index("c")
    s = lax.axis_index("s")
    wid = c * 16 + s
    base = wid * PER_W
    pltpu.sync_copy(idx_ref.at[pl.ds(base, PER_W)], idx_v)

    pltpu.async_copy(table_ref.at[idx_v.at[pl.ds(0, SCH)]],
                     rows_v.at[0], sems.at[0])

    def step(j, carry):
        b = j % 2
        nb = (j + 1) % 2

        @pl.when(j + 1 < NSTR)
        def _():
            pltpu.async_copy(
                table_ref.at[idx_v.at[pl.ds((j + 1) * SCH, SCH)]],
                rows_v.at[nb], sems.at[nb])

        pltpu.make_async_copy(
            table_ref.at[idx_v.at[pl.ds(j * SCH, SCH)]],
            rows_v.at[b], sems.at[b]).wait()
        pltpu.sync_copy(rows_v.at[b], out_ref.at[pl.ds(base + j * SCH, SCH)])
        return carry

    lax.fori_loop(0, NSTR, step, 0)


@functools.cache
def _make_gather():
    return pl.kernel(
        _gather_body,
        out_type=jax.ShapeDtypeStruct((NSAMP, 8), jnp.float32),
        mesh=plsc.VectorSubcoreMesh(core_axis_name="c", subcore_axis_name="s"),
        scratch_types=[
            pltpu.VMEM((PER_W,), jnp.int32),
            pltpu.VMEM((2, SCH, 8), jnp.float32),
            pltpu.SemaphoreType.DMA((2,)),
        ],
        compiler_params=pltpu.CompilerParams(use_tc_tiling_on_sc=False),
    )


def _scatter_body(vals_ref, idx_ref, zeros_ref, out_ref, idx_v, vals_v, accum,
                  sems):
    c = lax.axis_index("c")
    s = lax.axis_index("s")
    wid = c * 16 + s
    base = wid * PER_W

    @pl.when(s == 0)
    def _():
        pltpu.sync_copy(zeros_ref, accum)

    plsc.subcore_barrier()
    pltpu.sync_copy(idx_ref.at[pl.ds(base, PER_W)], idx_v)

    def step(j, carry):
        b = j % 2

        @pl.when(j >= 2)
        def _():
            pltpu.make_async_copy(
                vals_v.at[b],
                accum.at[idx_v.at[pl.ds((j - 2) * SCH, SCH)]],
                sems.at[b]).wait()

        pltpu.sync_copy(vals_ref.at[pl.ds(base + j * SCH, SCH)], vals_v.at[b])
        pltpu.async_copy(vals_v.at[b],
                         accum.at[idx_v.at[pl.ds(j * SCH, SCH)]],
                         sems.at[b], add=True)
        return carry

    lax.fori_loop(0, NSTR, step, 0)
    for j in (NSTR - 2, NSTR - 1):
        pltpu.make_async_copy(
            vals_v.at[j % 2],
            accum.at[idx_v.at[pl.ds(j * SCH, SCH)]],
            sems.at[j % 2]).wait()
    plsc.subcore_barrier()
    rpt = NRAYS // 16
    pltpu.sync_copy(accum.at[pl.ds(s * rpt, rpt)], out_ref.at[c, pl.ds(s * rpt, rpt)])


@functools.cache
def _make_scatter():
    return pl.kernel(
        _scatter_body,
        out_type=jax.ShapeDtypeStruct((2, NRAYS, 8), jnp.float32),
        mesh=plsc.VectorSubcoreMesh(core_axis_name="c", subcore_axis_name="s"),
        scratch_types=[
            pltpu.VMEM((PER_W,), jnp.int32),
            pltpu.VMEM((2, SCH, 8), jnp.float32),
            pltpu.VMEM_SHARED((NRAYS, 8), jnp.float32),
            pltpu.SemaphoreType.DMA((2,)),
        ],
        compiler_params=pltpu.CompilerParams(use_tc_tiling_on_sc=False),
    )


def _main_body(samT, dirs, depT, delT, ridxT, w1t, b1c, w2t, b2c,
               out_ref, ce_ref, cm_ref, cr_ref):
    pid = pl.program_id(0)

    @pl.when(pid == 0)
    def _():
        ce_ref[0, 0] = 0.0
        cm_ref[0, 0] = 0.0
        cr_ref[0, 0] = -1

    carry_e = ce_ref[0, 0]
    carry_m = cm_ref[0, 0]
    carry_r = cr_ref[0, 0]

    dT = dirs[...].T
    feat = jnp.concatenate([samT[...], dT[:3]], axis=0)
    h = lax.dot_general(w1t[...], feat, (((1,), (0,)), ((), ())),
                        preferred_element_type=jnp.float32) + b1c[...]
    h = jnp.maximum(h, 0.0)
    o = lax.dot_general(w2t[...], h, (((1,), (0,)), ((), ())),
                        preferred_element_type=jnp.float32) + b2c[...]
    color = 1.0 / (1.0 + jnp.exp(-o[:3]))
    x = o[3:4]
    sp = jnp.maximum(x, 0.0) + jnp.log(1.0 + jnp.exp(-jnp.abs(x)))
    tau = sp * delT[...]

    lanes = lax.broadcasted_iota(jnp.int32, (1, CB), 1)

    inc = tau
    d = 1
    while d < CB:
        sh = pltpu.roll(inc, d, 1)
        inc = inc + jnp.where(lanes >= d, sh, 0.0)
        d *= 2

    e_excl = (carry_e + inc) - tau

    rr = ridxT[...]
    rprev = pltpu.roll(rr, 1, 1)
    rprev = jnp.where(lanes == 0, carry_r, rprev)
    bnd = rr != rprev
    be = jnp.where(bnd, e_excl, 0.0)
    m = be
    d = 1
    while d < CB:
        sh = pltpu.roll(m, d, 1)
        m = jnp.maximum(m, jnp.where(lanes >= d, sh, 0.0))
        d *= 2
    m = jnp.maximum(m, carry_m)

    excl = e_excl - m
    w = jnp.exp(-excl) - jnp.exp(-(excl + tau))

    valsT = jnp.concatenate(
        [color * w, depT[...] * w, w, jnp.zeros((3, CB), jnp.float32)], axis=0)
    out_ref[...] = valsT.T

    lastm = lanes == (CB - 1)
    ce_ref[0, 0] = carry_e + jnp.sum(jnp.where(lastm, inc, 0.0))
    cm_ref[0, 0] = jnp.sum(jnp.where(lastm, m, 0.0))
    cr_ref[0, 0] = jnp.sum(jnp.where(lastm, rr, 0))


_main = pl.pallas_call(
    _main_body,
    grid=(GRID_B,),
    in_specs=[
        pl.BlockSpec((3, CB), lambda i: (0, i)),
        pl.BlockSpec((CB, 8), lambda i: (i, 0)),
        pl.BlockSpec((1, CB), lambda i: (0, i)),
        pl.BlockSpec((1, CB), lambda i: (0, i)),
        pl.BlockSpec((1, CB), lambda i: (0, i)),
        pl.BlockSpec((HID, 6), lambda i: (0, 0)),
        pl.BlockSpec((HID, 1), lambda i: (0, 0)),
        pl.BlockSpec((4, HID), lambda i: (0, 0)),
        pl.BlockSpec((4, 1), lambda i: (0, 0)),
    ],
    out_specs=pl.BlockSpec((CB, 8), lambda i: (i, 0)),
    out_shape=jax.ShapeDtypeStruct((NSAMP, 8), jnp.float32),
    scratch_shapes=[
        pltpu.SMEM((1, 1), jnp.float32),
        pltpu.SMEM((1, 1), jnp.float32),
        pltpu.SMEM((1, 1), jnp.int32),
    ],
)


def _compose_body(p0, p1, rgb_ref, dep_ref, alp_ref, hit_ref):
    S = p0[...] + p1[...]
    alpha = S[:, 4:5]
    rgb_ref[...] = (1.0 - alpha) + alpha * S[:, 0:3]
    dep_ref[...] = S[:, 3:4]
    alp_ref[...] = alpha
    hit_ref[...] = (alpha > 0.0).astype(jnp.int32)


RB = 2048
_compose = pl.pallas_call(
    _compose_body,
    grid=(NRAYS // RB,),
    in_specs=[
        pl.BlockSpec((RB, 8), lambda i: (i, 0)),
        pl.BlockSpec((RB, 8), lambda i: (i, 0)),
    ],
    out_specs=(
        pl.BlockSpec((RB, 3), lambda i: (i, 0)),
        pl.BlockSpec((RB, 1), lambda i: (i, 0)),
        pl.BlockSpec((RB, 1), lambda i: (i, 0)),
        pl.BlockSpec((RB, 1), lambda i: (i, 0)),
    ),
    out_shape=(
        jax.ShapeDtypeStruct((NRAYS, 3), jnp.float32),
        jax.ShapeDtypeStruct((NRAYS, 1), jnp.float32),
        jax.ShapeDtypeStruct((NRAYS, 1), jnp.float32),
        jax.ShapeDtypeStruct((NRAYS, 1), jnp.int32),
    ),
)


def kernel(rays_origins, rays_dirs, ridx, samples, depths, deltas, W1, b1, W2, b2):
    del rays_origins
    ridx32 = ridx.astype(jnp.int32)
    table = jnp.concatenate(
        [rays_dirs, jnp.zeros((NRAYS, 5), jnp.float32)], axis=1)
    dirs8 = _make_gather()(table, ridx32)

    vals = _main(samples.T, dirs8, depths.T, deltas.T,
                 ridx32.reshape(1, NSAMP), W1.T,
                 b1.reshape(HID, 1), W2.T, b2.reshape(4, 1))

    zer = jnp.zeros((NRAYS, 8), jnp.float32)
    parts = _make_scatter()(vals, ridx32, zer)

    rgb, dep, alp, hit32 = _compose(parts[0], parts[1])
    return (rgb, dep, alp, hit32[:, 0] != 0)
